# baseline (speedup 1.0000x reference)
import functools

import jax
import jax.numpy as jnp
from jax import lax
from jax.experimental import pallas as pl
from jax.experimental.pallas import tpu as pltpu
from jax.experimental.pallas import tpu_sc as plsc

DIM = 64
BATCH = 16384
WEIGHT_DECAY = 0.0001
LANES = 16
CHUNK = 128


def _make_sc_kernel(num_cores, num_subcores):
    nw = num_cores * num_subcores
    bpw = BATCH // nw
    n_chunks = bpw // CHUNK

    mesh = plsc.VectorSubcoreMesh(core_axis_name="c", subcore_axis_name="s")

    @functools.partial(
        pl.kernel,
        mesh=mesh,
        compiler_params=pltpu.CompilerParams(use_tc_tiling_on_sc=False),
        out_type=(
            jax.ShapeDtypeStruct((BATCH, LANES), jnp.float32),
            jax.ShapeDtypeStruct((nw, LANES), jnp.float32),
        ),
        scratch_types=[
            pltpu.VMEM((n_chunks, CHUNK), jnp.int32),
            pltpu.VMEM((n_chunks, CHUNK), jnp.int32),
            pltpu.VMEM((n_chunks, CHUNK), jnp.int32),
            pltpu.VMEM((bpw, DIM), jnp.float32),
            pltpu.VMEM((bpw, DIM), jnp.float32),
            pltpu.VMEM((bpw, DIM), jnp.float32),
            pltpu.VMEM((bpw, LANES), jnp.float32),
            pltpu.VMEM((LANES,), jnp.float32),
            pltpu.SemaphoreType.DMA,
        ],
    )
    def sc_kernel(u_hbm, i_hbm, j_hbm, w_hbm, h_hbm, x_hbm, ssq_hbm,
                  uidx, iidx, jidx, urows, irows, jrows, xv, sqv, sem):
        wid = lax.axis_index("s") * num_cores + lax.axis_index("c")
        base = wid * bpw

        pltpu.sync_copy(u_hbm.at[pl.ds(wid * n_chunks, n_chunks)], uidx)
        pltpu.sync_copy(i_hbm.at[pl.ds(wid * n_chunks, n_chunks)], iidx)
        pltpu.sync_copy(j_hbm.at[pl.ds(wid * n_chunks, n_chunks)], jidx)

        descs = []
        for c in range(n_chunks):
            sl = pl.ds(c * CHUNK, CHUNK)
            descs.append(pltpu.async_copy(w_hbm.at[uidx.at[c]], urows.at[sl], sem))
            descs.append(pltpu.async_copy(h_hbm.at[iidx.at[c]], irows.at[sl], sem))
            descs.append(pltpu.async_copy(h_hbm.at[jidx.at[c]], jrows.at[sl], sem))
        for d in descs:
            d.wait()

        zero = jnp.zeros((LANES,), jnp.float32)

        def body(s, sq):
            ur = urows.at[s]
            ir = irows.at[s]
            jr = jrows.at[s]
            acc = zero
            for k in range(DIM // LANES):
                sl = pl.ds(k * LANES, LANES)
                uv = ur[sl]
                iv = ir[sl]
                jv = jr[sl]
                acc = acc + uv * (iv - jv)
                sq = sq + uv * uv + iv * iv + jv * jv
            xv[s] = acc
            return sq

        sq = lax.fori_loop(0, bpw, body, zero)
        sqv[...] = sq

        pltpu.sync_copy(xv, x_hbm.at[pl.ds(base, bpw)])
        pltpu.sync_copy(sqv, ssq_hbm.at[wid])

    return sc_kernel


def _tc_reduce(x_ref, ssq_ref, o_ref):
    x = jnp.sum(x_ref[...], axis=1)
    ls = jnp.minimum(x, 0.0) - jnp.log1p(jnp.exp(-jnp.abs(x)))
    o_ref[0, 0] = -jnp.sum(ls) + WEIGHT_DECAY * jnp.sum(ssq_ref[...])


def kernel(u, i, j, W, H):
    info = plsc.get_sparse_core_info()
    sc_fn = _make_sc_kernel(info.num_cores, info.num_subcores)

    shape2d = (BATCH // CHUNK, CHUNK)
    x, ssq = sc_fn(
        u.astype(jnp.int32).reshape(shape2d),
        i.astype(jnp.int32).reshape(shape2d),
        j.astype(jnp.int32).reshape(shape2d),
        W,
        H,
    )

    loss = pl.pallas_call(
        _tc_reduce,
        out_shape=jax.ShapeDtypeStruct((1, 1), jnp.float32),
        out_specs=pl.BlockSpec(memory_space=pltpu.SMEM),
    )(x, ssq)
    return loss[0, 0]


# speedup vs baseline: 1.0083x; 1.0083x over previous
import functools

import jax
import jax.numpy as jnp
from jax import lax
from jax.experimental import pallas as pl
from jax.experimental.pallas import tpu as pltpu
from jax.experimental.pallas import tpu_sc as plsc

DIM = 64
BATCH = 16384
WEIGHT_DECAY = 0.0001
LANES = 16
CHUNK = 128
ROW = 128


def _make_sc_kernel(num_cores, num_subcores):
    nw = num_cores * num_subcores
    bpw = BATCH // nw
    n_chunks = bpw // CHUNK

    mesh = plsc.VectorSubcoreMesh(core_axis_name="c", subcore_axis_name="s")

    @functools.partial(
        pl.kernel,
        mesh=mesh,
        out_type=(
            jax.ShapeDtypeStruct((BATCH * LANES,), jnp.float32),
            jax.ShapeDtypeStruct((nw * LANES,), jnp.float32),
        ),
        scratch_types=[
            pltpu.VMEM((bpw,), jnp.int32),
            pltpu.VMEM((bpw,), jnp.int32),
            pltpu.VMEM((bpw,), jnp.int32),
            pltpu.VMEM((bpw,), jnp.int32),
            pltpu.VMEM((bpw,), jnp.int32),
            pltpu.VMEM((bpw,), jnp.int32),
            pltpu.VMEM((2, CHUNK, ROW), jnp.float32),
            pltpu.VMEM((2, CHUNK, ROW), jnp.float32),
            pltpu.VMEM((2, CHUNK, ROW), jnp.float32),
            pltpu.VMEM((bpw * LANES,), jnp.float32),
            pltpu.VMEM((LANES,), jnp.float32),
            pltpu.SemaphoreType.DMA,
        ],
    )
    def sc_kernel(uh_hbm, ih_hbm, jh_hbm, uo_hbm, io_hbm, jo_hbm, w2_hbm, h2_hbm,
                  x_hbm, ssq_hbm,
                  uhx, ihx, jhx, uos, ios, jos, ub, ib, jb, xv, sqv, sem):
        wid = lax.axis_index("s") * num_cores + lax.axis_index("c")
        base = wid * bpw

        pltpu.sync_copy(uh_hbm.at[pl.ds(base, bpw)], uhx)
        pltpu.sync_copy(ih_hbm.at[pl.ds(base, bpw)], ihx)
        pltpu.sync_copy(jh_hbm.at[pl.ds(base, bpw)], jhx)
        pltpu.sync_copy(uo_hbm.at[pl.ds(base, bpw)], uos)
        pltpu.sync_copy(io_hbm.at[pl.ds(base, bpw)], ios)
        pltpu.sync_copy(jo_hbm.at[pl.ds(base, bpw)], jos)

        def fire(c):
            b = c % 2
            sl = pl.ds(c * CHUNK, CHUNK)
            pltpu.async_copy(w2_hbm.at[uhx.at[sl]], ub.at[b], sem)
            pltpu.async_copy(h2_hbm.at[ihx.at[sl]], ib.at[b], sem)
            pltpu.async_copy(h2_hbm.at[jhx.at[sl]], jb.at[b], sem)

        def drain():
            pltpu.make_async_copy(w2_hbm.at[pl.ds(0, CHUNK)], ub.at[0], sem).wait()
            pltpu.make_async_copy(h2_hbm.at[pl.ds(0, CHUNK)], ib.at[0], sem).wait()
            pltpu.make_async_copy(h2_hbm.at[pl.ds(0, CHUNK)], jb.at[0], sem).wait()

        fire(0)
        zero = jnp.zeros((LANES,), jnp.float32)
        sq = zero
        for c in range(n_chunks):
            b = c % 2
            drain()
            if c + 1 < n_chunks:
                fire(c + 1)

            def body(g, sq):
                gb = g * LANES
                ou16 = uos[pl.ds(c * CHUNK + gb, LANES)]
                oi16 = ios[pl.ds(c * CHUNK + gb, LANES)]
                oj16 = jos[pl.ds(c * CHUNK + gb, LANES)]
                for s in range(LANES):
                    ur = ub.at[b, gb + s]
                    ir = ib.at[b, gb + s]
                    jr = jb.at[b, gb + s]
                    ou = ou16[s]
                    oi = oi16[s]
                    oj = oj16[s]
                    acc = zero
                    for k in range(DIM // LANES):
                        uv = ur[pl.ds(ou + k * LANES, LANES)]
                        iv = ir[pl.ds(oi + k * LANES, LANES)]
                        jv = jr[pl.ds(oj + k * LANES, LANES)]
                        acc = acc + uv * (iv - jv)
                        sq = sq + uv * uv + iv * iv + jv * jv
                    xv[pl.ds((c * CHUNK + gb + s) * LANES, LANES)] = acc
                return sq

            sq = lax.fori_loop(0, CHUNK // LANES, body, sq)

        sqv[...] = sq
        pltpu.sync_copy(xv, x_hbm.at[pl.ds(base * LANES, bpw * LANES)])
        pltpu.sync_copy(sqv, ssq_hbm.at[pl.ds(wid * LANES, LANES)])

    return sc_kernel


def _tc_reduce(x_ref, ssq_ref, o_ref):
    x = x_ref[...]
    lane = lax.broadcasted_iota(jnp.int32, (128, 8), 0)
    grp = lax.broadcasted_iota(jnp.int32, (128, 8), 1)
    sel = jnp.where(lane // LANES == grp, 1.0, 0.0).astype(jnp.float32)
    xs = jax.lax.dot_general(x, sel, (((1,), (0,)), ((), ())),
                             preferred_element_type=jnp.float32)
    ls = jnp.minimum(xs, 0.0) - jnp.log1p(jnp.exp(-jnp.abs(xs)))
    o_ref[0, 0] = -jnp.sum(ls) + WEIGHT_DECAY * jnp.sum(ssq_ref[...])


def kernel(u, i, j, W, H):
    info = plsc.get_sparse_core_info()
    sc_fn = _make_sc_kernel(info.num_cores, info.num_subcores)

    u = u.astype(jnp.int32)
    i = i.astype(jnp.int32)
    j = j.astype(jnp.int32)
    x, ssq = sc_fn(
        u >> 1, i >> 1, j >> 1,
        (u & 1) << 6, (i & 1) << 6, (j & 1) << 6,
        W.reshape(W.shape[0] // 2, 2 * DIM),
        H.reshape(H.shape[0] // 2, 2 * DIM),
    )

    loss = pl.pallas_call(
        _tc_reduce,
        out_shape=jax.ShapeDtypeStruct((1, 1), jnp.float32),
        out_specs=pl.BlockSpec(memory_space=pltpu.SMEM),
    )(x.reshape(BATCH * LANES // 128, 128), ssq.reshape(4, 128))
    return loss[0, 0]


# speedup vs baseline: 1.0793x; 1.0704x over previous
import functools

import jax
import jax.numpy as jnp
from jax import lax
from jax.experimental import pallas as pl
from jax.experimental.pallas import tpu as pltpu
from jax.experimental.pallas import tpu_sc as plsc

DIM = 64
BATCH = 16384
WEIGHT_DECAY = 0.0001
LANES = 16
CHUNK = 128
ROW = 128


def _make_sc_kernel(num_cores, num_subcores):
    nw = num_cores * num_subcores
    bpw = BATCH // nw
    n_chunks = bpw // CHUNK

    mesh = plsc.VectorSubcoreMesh(core_axis_name="c", subcore_axis_name="s")

    @functools.partial(
        pl.kernel,
        mesh=mesh,
        out_type=(
            jax.ShapeDtypeStruct((BATCH * LANES,), jnp.float32),
            jax.ShapeDtypeStruct((nw * LANES,), jnp.float32),
        ),
        scratch_types=[
            pltpu.VMEM((bpw,), jnp.int32),
            pltpu.VMEM((bpw,), jnp.int32),
            pltpu.VMEM((bpw,), jnp.int32),
            pltpu.VMEM((2, CHUNK, ROW), jnp.float32),
            pltpu.VMEM((2, CHUNK, ROW), jnp.float32),
            pltpu.VMEM((2, CHUNK, ROW), jnp.float32),
            pltpu.VMEM((bpw * LANES,), jnp.float32),
            pltpu.VMEM((LANES,), jnp.float32),
            pltpu.SemaphoreType.DMA,
        ],
    )
    def sc_kernel(u_hbm, i_hbm, j_hbm, w2_hbm, h2_hbm, x_hbm, ssq_hbm,
                  ux, ix, jx, ub, ib, jb, xv, sqv, sem):
        wid = lax.axis_index("s") * num_cores + lax.axis_index("c")
        base = wid * bpw

        pltpu.sync_copy(u_hbm.at[pl.ds(base, bpw)], ux)
        pltpu.sync_copy(i_hbm.at[pl.ds(base, bpw)], ix)
        pltpu.sync_copy(j_hbm.at[pl.ds(base, bpw)], jx)

        def fire(c):
            b = c % 2
            sl = pl.ds(c * CHUNK, CHUNK)
            pltpu.async_copy(w2_hbm.at[ux.at[sl]], ub.at[b], sem)
            pltpu.async_copy(h2_hbm.at[ix.at[sl]], ib.at[b], sem)
            pltpu.async_copy(h2_hbm.at[jx.at[sl]], jb.at[b], sem)

        def drain():
            pltpu.make_async_copy(w2_hbm.at[pl.ds(0, CHUNK)], ub.at[0], sem).wait()
            pltpu.make_async_copy(h2_hbm.at[pl.ds(0, CHUNK)], ib.at[0], sem).wait()
            pltpu.make_async_copy(h2_hbm.at[pl.ds(0, CHUNK)], jb.at[0], sem).wait()

        fire(0)
        zero = jnp.zeros((LANES,), jnp.float32)
        sq = zero
        for c in range(n_chunks):
            b = c % 2
            drain()
            if c + 1 < n_chunks:
                fire(c + 1)

            def body(s, sq):
                ur = ub.at[b, s]
                ir = ib.at[b, s]
                jr = jb.at[b, s]
                acc = zero
                for k in range(DIM // LANES):
                    uv = ur[pl.ds(k * LANES, LANES)]
                    iv = ir[pl.ds(k * LANES, LANES)]
                    jv = jr[pl.ds(k * LANES, LANES)]
                    acc = acc + uv * (iv - jv)
                    sq = sq + uv * uv + iv * iv + jv * jv
                xv[pl.ds((c * CHUNK + s) * LANES, LANES)] = acc
                return sq

            sq = lax.fori_loop(0, CHUNK, body, sq)

        sqv[...] = sq
        pltpu.sync_copy(xv, x_hbm.at[pl.ds(base * LANES, bpw * LANES)])
        pltpu.sync_copy(sqv, ssq_hbm.at[pl.ds(wid * LANES, LANES)])

    return sc_kernel


def _tc_reduce(x_ref, ssq_ref, o_ref):
    x = x_ref[...]
    lane = lax.broadcasted_iota(jnp.int32, (128, 8), 0)
    grp = lax.broadcasted_iota(jnp.int32, (128, 8), 1)
    sel = jnp.where(lane // LANES == grp, 1.0, 0.0).astype(jnp.float32)
    xs = jax.lax.dot_general(x, sel, (((1,), (0,)), ((), ())),
                             preferred_element_type=jnp.float32)
    ls = jnp.minimum(xs, 0.0) - jnp.log1p(jnp.exp(-jnp.abs(xs)))
    o_ref[0, 0] = -jnp.sum(ls) + WEIGHT_DECAY * jnp.sum(ssq_ref[...])


def kernel(u, i, j, W, H):
    info = plsc.get_sparse_core_info()
    sc_fn = _make_sc_kernel(info.num_cores, info.num_subcores)

    x, ssq = sc_fn(
        u.astype(jnp.int32),
        i.astype(jnp.int32),
        j.astype(jnp.int32),
        jnp.pad(W, ((0, 0), (0, ROW - DIM))),
        jnp.pad(H, ((0, 0), (0, ROW - DIM))),
    )

    loss = pl.pallas_call(
        _tc_reduce,
        out_shape=jax.ShapeDtypeStruct((1, 1), jnp.float32),
        out_specs=pl.BlockSpec(memory_space=pltpu.SMEM),
    )(x.reshape(BATCH * LANES // 128, 128), ssq.reshape(4, 128))
    return loss[0, 0]
